# baseline (speedup 1.0000x reference)
import functools

import jax
import jax.numpy as jnp
from jax import lax
from jax.experimental import pallas as pl
from jax.experimental.pallas import tpu as pltpu
from jax.experimental.pallas import tpu_sc as plsc

_L = 16


def _sc_add_kernel(batch, seq_len, d_model, num_workers, chunk_rows):
    rows_per_worker = seq_len // num_workers
    num_chunks = rows_per_worker // chunk_rows
    cd = chunk_rows * d_model
    num_vecs = cd // _L

    mesh = plsc.VectorSubcoreMesh(core_axis_name="c", subcore_axis_name="s")

    @functools.partial(
        pl.kernel,
        mesh=mesh,
        out_type=jax.ShapeDtypeStruct((batch, seq_len * d_model), jnp.float32),
        scratch_types=[
            pltpu.VMEM((cd,), jnp.float32),
            pltpu.VMEM((cd,), jnp.float32),
        ],
    )
    def k(x_hbm, pe_hbm, out_hbm, pe_v, x_v):
        nc = 2
        wid = lax.axis_index("s") * nc + lax.axis_index("c")
        worker_base = wid * (rows_per_worker * d_model)

        def chunk_body(ci, _):
            base = worker_base + ci * cd
            pltpu.sync_copy(pe_hbm.at[pl.ds(base, cd)], pe_v)
            for b in range(batch):
                pltpu.sync_copy(x_hbm.at[b, pl.ds(base, cd)], x_v)

                def add_body(i, _):
                    sl = pl.ds(i * _L, _L)
                    x_v[sl] = x_v[sl] + pe_v[sl]
                    return _

                lax.fori_loop(0, num_vecs, add_body, None, unroll=8)
                pltpu.sync_copy(x_v, out_hbm.at[b, pl.ds(base, cd)])
            return _

        lax.fori_loop(0, num_chunks, chunk_body, None)

    return k


def kernel(x, pe_weight):
    batch, seq_len, d_model = x.shape
    x2 = x.reshape(batch, seq_len * d_model)
    pe2 = pe_weight.reshape(-1)
    out = _sc_add_kernel(batch, seq_len, d_model, 32, 16)(x2, pe2)
    return out.reshape(batch, seq_len, d_model)


# speedup vs baseline: 1.0864x; 1.0864x over previous
import functools

import jax
import jax.numpy as jnp
from jax import lax
from jax.experimental import pallas as pl
from jax.experimental.pallas import tpu as pltpu
from jax.experimental.pallas import tpu_sc as plsc

_L = 16


def _sc_add_kernel(batch, seq_len, d_model, num_workers, chunk_rows):
    rows_per_worker = seq_len // num_workers
    nchunks = rows_per_worker // chunk_rows
    cd = chunk_rows * d_model
    num_vecs = cd // _L

    mesh = plsc.VectorSubcoreMesh(core_axis_name="c", subcore_axis_name="s")

    @functools.partial(
        pl.kernel,
        mesh=mesh,
        out_type=jax.ShapeDtypeStruct((batch, seq_len * d_model), jnp.float32),
        scratch_types=[
            pltpu.VMEM((2, cd), jnp.float32),
            pltpu.VMEM((2 * batch, cd), jnp.float32),
            pltpu.SemaphoreType.DMA((2,)),
            pltpu.SemaphoreType.DMA((2 * batch,)),
            pltpu.SemaphoreType.DMA((2 * batch,)),
        ],
    )
    def k(x_hbm, pe_hbm, out_hbm, pe_v, x_v, pe_sem, in_sem, out_sem):
        nc = 2
        wid = lax.axis_index("s") * nc + lax.axis_index("c")
        worker_base = wid * (rows_per_worker * d_model)

        def pe_copy(ci, slot):
            base = worker_base + ci * cd
            return pltpu.make_async_copy(
                pe_hbm.at[pl.ds(base, cd)], pe_v.at[slot], pe_sem.at[slot]
            )

        def in_copy(ci, b, slot):
            base = worker_base + ci * cd
            return pltpu.make_async_copy(
                x_hbm.at[b, pl.ds(base, cd)], x_v.at[slot], in_sem.at[slot]
            )

        def out_copy(ci, b, slot):
            base = worker_base + ci * cd
            return pltpu.make_async_copy(
                x_v.at[slot], out_hbm.at[b, pl.ds(base, cd)], out_sem.at[slot]
            )

        pe_copy(0, 0).start()
        for b in range(batch):
            in_copy(0, b, b).start()

        def chunk_pair(ci2, _):
            for p in range(2):
                ci = ci2 * 2 + p

                def pe_prefetch():
                    pe_copy(ci + 1, 1 - p).start()

                if p == 0:
                    pe_prefetch()
                else:
                    pl.when(ci2 < nchunks // 2 - 1)(pe_prefetch)
                pe_copy(ci, p).wait()

                for b in range(batch):
                    cur = p * batch + b
                    nxt = (1 - p) * batch + b

                    def refill():
                        def drain_prev():
                            out_copy(ci - 1, b, nxt).wait()

                        if p == 1:
                            drain_prev()
                        else:
                            pl.when(ci2 >= 1)(drain_prev)
                        in_copy(ci + 1, b, nxt).start()

                    if p == 0:
                        refill()
                    else:
                        pl.when(ci2 < nchunks // 2 - 1)(refill)

                    in_copy(ci, b, cur).wait()

                    def add_body(i, carry):
                        sl = pl.ds(i * _L, _L)
                        x_v[cur, sl] = x_v[cur, sl] + pe_v[p, sl]
                        return carry

                    lax.fori_loop(0, num_vecs, add_body, None, unroll=8)
                    out_copy(ci, b, cur).start()
            return _

        lax.fori_loop(0, nchunks // 2, chunk_pair, None)

        for p in range(2):
            ci = nchunks - 2 + p
            for b in range(batch):
                out_copy(ci, b, p * batch + b).wait()

    return k


def kernel(x, pe_weight):
    batch, seq_len, d_model = x.shape
    x2 = x.reshape(batch, seq_len * d_model)
    pe2 = pe_weight.reshape(-1)
    out = _sc_add_kernel(batch, seq_len, d_model, 32, 8)(x2, pe2)
    return out.reshape(batch, seq_len, d_model)
